# baseline (speedup 1.0000x reference)
import functools

import jax
import jax.numpy as jnp
from jax import lax
from jax.experimental import pallas as pl
from jax.experimental.pallas import tpu as pltpu
from jax.experimental.pallas import tpu_sc as plsc

B, S, D = 4, 2048, 1024
T = B * S
CH = 8
NGRP = 3
LANES = 16


def _build_kernel():
    info = plsc.get_sparse_core_info()
    nc, ns = info.num_cores, info.num_subcores
    nw = nc * ns
    p_per_w = S // nw
    n_q = p_per_w // CH
    rows_g = B * CH

    @functools.partial(
        pl.kernel,
        mesh=plsc.VectorSubcoreMesh(core_axis_name="c", subcore_axis_name="s"),
        out_type=jax.ShapeDtypeStruct((T, D), jnp.float32),
        scratch_types=(
            [pltpu.VMEM((B, p_per_w), jnp.int32)]
            + [pltpu.VMEM((rows_g, D), jnp.float32) for _ in range(NGRP)]
            + [pltpu.VMEM((CH, D), jnp.float32) for _ in range(NGRP)]
            + [pltpu.SemaphoreType.DMA for _ in range(3 * NGRP)]
        ),
    )
    def emb_kernel(ids_hbm, tok_hbm, pos_hbm, out_hbm,
                   idx_v, tb0, tb1, tb2, pb0, pb1, pb2,
                   g0, g1, g2, p0s, p1s, p2s, s0, s1, s2):
        cid = lax.axis_index("c")
        sid = lax.axis_index("s")
        wid = sid * nc + cid
        p0 = wid * p_per_w

        tbufs = (tb0, tb1, tb2)
        pbufs = (pb0, pb1, pb2)
        gsems = (g0, g1, g2)
        psems = (p0s, p1s, p2s)
        ssems = (s0, s1, s2)

        for b in range(B):
            pltpu.sync_copy(
                ids_hbm.at[b, pl.ds(pl.multiple_of(p0, 8), p_per_w)],
                idx_v.at[b])

        pend_in = [None] * NGRP
        pend_store = [None] * NGRP

        def issue_chunk(q):
            g = q % NGRP
            if pend_store[g] is not None:
                for h in pend_store[g]:
                    h.wait()
                pend_store[g] = None
            hp = pltpu.make_async_copy(
                pos_hbm.at[pl.ds(pl.multiple_of(p0 + q * CH, 8), CH), :],
                pbufs[g], psems[g])
            hp.start()
            hs = [hp]
            for b in range(B):
                hg = pltpu.make_async_copy(
                    tok_hbm.at[idx_v.at[b, pl.ds(q * CH, CH)]],
                    tbufs[g].at[pl.ds(b * CH, CH)], gsems[g])
                hg.start()
                hs.append(hg)
            pend_in[g] = tuple(hs)

        for q in range(NGRP):
            issue_chunk(q)

        for q in range(n_q):
            g = q % NGRP
            for h in pend_in[g]:
                h.wait()
            pend_in[g] = None

            pb = pbufs[g]
            tb = tbufs[g]

            def row_body(r, carry, _pb=pb, _tb=tb):
                for c in range(D // LANES):
                    v = _pb[r, pl.ds(c * LANES, LANES)]
                    for b in range(B):
                        plsc.addupdate(
                            _tb.at[b * CH + r, pl.ds(c * LANES, LANES)], v)
                return carry

            lax.fori_loop(0, CH, row_body, 0)

            hs = []
            for b in range(B):
                h = pltpu.make_async_copy(
                    tb.at[pl.ds(b * CH, CH)],
                    out_hbm.at[pl.ds(b * S + p0 + q * CH, CH), :],
                    ssems[g])
                h.start()
                hs.append(h)
            pend_store[g] = hs

            if q + NGRP < n_q:
                issue_chunk(q + NGRP)

        for g in range(NGRP):
            if pend_store[g] is not None:
                for h in pend_store[g]:
                    h.wait()

    return emb_kernel, n_q, rows_g, nw


_BUILT = None


def kernel(input_ids, token_table, pos_table):
    global _BUILT
    if _BUILT is None:
        _BUILT = _build_kernel()
    emb_kernel, n_q, rows_g, nw = _BUILT
    ids = input_ids.astype(jnp.int32)
    out = emb_kernel(ids, token_table, pos_table)
    return out.reshape(B, S, D)


# speedup vs baseline: 1.0117x; 1.0117x over previous
import functools

import jax
import jax.numpy as jnp
from jax import lax
from jax.experimental import pallas as pl
from jax.experimental.pallas import tpu as pltpu
from jax.experimental.pallas import tpu_sc as plsc

B, S, D = 4, 2048, 1024
T = B * S
CH = 8
NGRP = 3
LANES = 16


def _build_kernel():
    info = plsc.get_sparse_core_info()
    nc, ns = info.num_cores, info.num_subcores
    nw = nc * ns
    p_per_w = S // nw
    n_q = p_per_w // CH
    rows_g = B * CH

    @functools.partial(
        pl.kernel,
        mesh=plsc.VectorSubcoreMesh(core_axis_name="c", subcore_axis_name="s"),
        out_type=jax.ShapeDtypeStruct((T, D), jnp.float32),
        scratch_types=(
            [pltpu.VMEM((n_q, rows_g), jnp.int32)]
            + [pltpu.VMEM((rows_g, D), jnp.float32) for _ in range(NGRP)]
            + [pltpu.VMEM((CH, D), jnp.float32) for _ in range(NGRP)]
            + [pltpu.SemaphoreType.DMA for _ in range(3 * NGRP)]
        ),
    )
    def emb_kernel(ids_hbm, tok_hbm, pos_hbm, out_hbm,
                   idx_v, tb0, tb1, tb2, pb0, pb1, pb2,
                   g0, g1, g2, p0s, p1s, p2s, s0, s1, s2):
        cid = lax.axis_index("c")
        sid = lax.axis_index("s")
        wid = sid * nc + cid
        p0 = wid * p_per_w

        tbufs = (tb0, tb1, tb2)
        pbufs = (pb0, pb1, pb2)
        gsems = (g0, g1, g2)
        psems = (p0s, p1s, p2s)
        ssems = (s0, s1, s2)

        pltpu.sync_copy(
            ids_hbm.at[pl.ds(pl.multiple_of(wid * n_q, 8), n_q), :], idx_v)

        pend_in = [None] * NGRP
        pend_store = [None] * NGRP

        def issue_chunk(q):
            g = q % NGRP
            if pend_store[g] is not None:
                for h in pend_store[g]:
                    h.wait()
                pend_store[g] = None
            hp = pltpu.make_async_copy(
                pos_hbm.at[pl.ds(pl.multiple_of(p0 + q * CH, 8), CH), :],
                pbufs[g], psems[g])
            hp.start()
            hg = pltpu.make_async_copy(
                tok_hbm.at[idx_v.at[q]], tbufs[g], gsems[g])
            hg.start()
            pend_in[g] = (hp, hg)

        for q in range(NGRP):
            issue_chunk(q)

        for q in range(n_q):
            g = q % NGRP
            for h in pend_in[g]:
                h.wait()
            pend_in[g] = None

            pb = pbufs[g]
            tb = tbufs[g]

            def row_body(r, carry, _pb=pb, _tb=tb):
                def blk_body(cb, inner, _pb=_pb, _tb=_tb, _r=r):
                    for ci in range(8):
                        off = cb * (8 * LANES) + ci * LANES
                        v = _pb[_r, pl.ds(off, LANES)]
                        for b in range(B):
                            plsc.addupdate(
                                _tb.at[b * CH + _r, pl.ds(off, LANES)], v)
                    return inner

                return lax.fori_loop(0, D // (8 * LANES), blk_body, carry)

            lax.fori_loop(0, CH, row_body, 0)

            hs = []
            for b in range(B):
                h = pltpu.make_async_copy(
                    tb.at[pl.ds(b * CH, CH)],
                    out_hbm.at[pl.ds(b * S + p0 + q * CH, CH), :],
                    ssems[g])
                h.start()
                hs.append(h)
            pend_store[g] = hs

            if q + NGRP < n_q:
                issue_chunk(q + NGRP)

        for g in range(NGRP):
            if pend_store[g] is not None:
                for h in pend_store[g]:
                    h.wait()

    return emb_kernel, n_q, rows_g, nw


_BUILT = None


def kernel(input_ids, token_table, pos_table):
    global _BUILT
    if _BUILT is None:
        _BUILT = _build_kernel()
    emb_kernel, n_q, rows_g, nw = _BUILT
    ids = input_ids.astype(jnp.int32).reshape(B, nw, n_q, CH)
    ids = ids.transpose(1, 2, 0, 3).reshape(nw * n_q, rows_g)
    out = emb_kernel(ids, token_table, pos_table)
    return out.reshape(B, S, D)


# speedup vs baseline: 1.0213x; 1.0095x over previous
import functools

import jax
import jax.numpy as jnp
from jax import lax
from jax.experimental import pallas as pl
from jax.experimental.pallas import tpu as pltpu
from jax.experimental.pallas import tpu_sc as plsc

B, S, D = 4, 2048, 1024
T = B * S
CH = 8
NGRP = 3
LANES = 16


def _build_kernel():
    info = plsc.get_sparse_core_info()
    nc, ns = info.num_cores, info.num_subcores
    nw = nc * ns
    p_per_w = S // nw
    n_q = p_per_w // CH
    rows_g = B * CH

    @functools.partial(
        pl.kernel,
        mesh=plsc.VectorSubcoreMesh(core_axis_name="c", subcore_axis_name="s"),
        out_type=jax.ShapeDtypeStruct((T, D), jnp.float32),
        scratch_types=(
            [pltpu.VMEM((n_q, rows_g), jnp.int32)]
            + [pltpu.VMEM((rows_g, D), jnp.float32) for _ in range(NGRP)]
            + [pltpu.VMEM((CH, D), jnp.float32) for _ in range(NGRP)]
            + [pltpu.SemaphoreType.DMA for _ in range(3 * NGRP)]
        ),
    )
    def emb_kernel(ids_hbm, tok_hbm, pos_hbm, out_hbm,
                   idx_v, tb0, tb1, tb2, pb0, pb1, pb2,
                   g0, g1, g2, p0s, p1s, p2s, s0, s1, s2):
        cid = lax.axis_index("c")
        sid = lax.axis_index("s")
        wid = sid * nc + cid
        p0 = wid * p_per_w

        tbufs = (tb0, tb1, tb2)
        pbufs = (pb0, pb1, pb2)
        gsems = (g0, g1, g2)
        psems = (p0s, p1s, p2s)
        ssems = (s0, s1, s2)

        pltpu.sync_copy(
            ids_hbm.at[pl.ds(pl.multiple_of(wid * n_q, 8), n_q), :], idx_v)

        pend_in = [None] * NGRP
        pend_store = [None] * NGRP

        def issue_chunk(q):
            g = q % NGRP
            if pend_store[g] is not None:
                for h in pend_store[g]:
                    h.wait()
                pend_store[g] = None
            hp = pltpu.make_async_copy(
                pos_hbm.at[pl.ds(pl.multiple_of(p0 + q * CH, 8), CH), :],
                pbufs[g], psems[g])
            hp.start()
            hg = pltpu.make_async_copy(
                tok_hbm.at[idx_v.at[q]], tbufs[g], gsems[g])
            hg.start()
            pend_in[g] = (hp, hg)

        for q in range(NGRP):
            issue_chunk(q)

        for q in range(n_q):
            g = q % NGRP
            for h in pend_in[g]:
                h.wait()
            pend_in[g] = None

            pb = pbufs[g]
            tb = tbufs[g]

            def row_body(r, carry, _pb=pb, _tb=tb):
                for c in range(D // LANES):
                    v = _pb[r, pl.ds(c * LANES, LANES)]
                    for b in range(B):
                        plsc.addupdate(
                            _tb.at[b * CH + r, pl.ds(c * LANES, LANES)], v)
                return carry

            lax.fori_loop(0, CH, row_body, 0)

            hs = []
            for b in range(B):
                h = pltpu.make_async_copy(
                    tb.at[pl.ds(b * CH, CH)],
                    out_hbm.at[pl.ds(b * S + p0 + q * CH, CH), :],
                    ssems[g])
                h.start()
                hs.append(h)
            pend_store[g] = hs

            if q + NGRP < n_q:
                issue_chunk(q + NGRP)

        for g in range(NGRP):
            if pend_store[g] is not None:
                for h in pend_store[g]:
                    h.wait()

    return emb_kernel, n_q, rows_g, nw


_BUILT = None


def kernel(input_ids, token_table, pos_table):
    global _BUILT
    if _BUILT is None:
        _BUILT = _build_kernel()
    emb_kernel, n_q, rows_g, nw = _BUILT
    ids = input_ids.astype(jnp.int32).reshape(B, nw, n_q, CH)
    ids = ids.transpose(1, 2, 0, 3).reshape(nw * n_q, rows_g)
    out = emb_kernel(ids, token_table, pos_table)
    return out.reshape(B, S, D)
